# Initial kernel scaffold; baseline (speedup 1.0000x reference)
import jax
import jax.numpy as jnp
from jax.experimental import pallas as pl


def kernel(x, edge_index, in_w, in_b, ggnn_w, wih, whh, bih, bhh, ln_g, ln_b):
    raise NotImplementedError("write your pallas kernel here")


# speedup vs baseline: 3.9621x; 3.9621x over previous
import functools

import jax
import jax.numpy as jnp
from jax import lax
from jax.experimental import pallas as pl
from jax.experimental.pallas import tpu as pltpu
from jax.experimental.pallas import tpu_sc as plsc

_NC = 2
_NS = 16
_CHUNK = 80


@functools.partial(jax.jit, static_argnames=())
def _sc_aggregate(m, src, dst, zrows):
    N, H = m.shape
    E = src.shape[0]
    n_workers = _NC * _NS
    ew = E // n_workers
    n_chunks = ew // _CHUNK
    rpt = (N // _NS) // 8 * 8
    tail = N - _NS * rpt

    mesh = plsc.VectorSubcoreMesh(core_axis_name="c", subcore_axis_name="s")

    @functools.partial(
        pl.kernel,
        mesh=mesh,
        out_type=jax.ShapeDtypeStruct((2 * N, H), jnp.float32),
        scratch_types=[
            pltpu.VMEM((_CHUNK,), jnp.int32),
            pltpu.VMEM((_CHUNK,), jnp.int32),
            pltpu.VMEM((_CHUNK, H), jnp.float32),
            pltpu.VMEM_SHARED((N, H), jnp.float32),
            pltpu.SemaphoreType.DMA,
        ],
    )
    def sc_fn(m_hbm, src_hbm, dst_hbm, z_hbm, out_hbm, src_v, dst_v, rows_v,
              agg_sh, sem):
        cid = lax.axis_index("c")
        sid = lax.axis_index("s")
        wid = sid * _NC + cid
        pltpu.sync_copy(z_hbm, agg_sh.at[pl.ds(sid * rpt, rpt)])
        if tail:
            @pl.when(sid == 0)
            def _():
                pltpu.sync_copy(z_hbm.at[pl.ds(0, tail)],
                                agg_sh.at[pl.ds(_NS * rpt, tail)])
        plsc.subcore_barrier()

        def body(i, carry):
            base = wid * ew + i * _CHUNK
            pltpu.sync_copy(src_hbm.at[pl.ds(base, _CHUNK)], src_v)
            pltpu.sync_copy(dst_hbm.at[pl.ds(base, _CHUNK)], dst_v)
            pltpu.async_copy(m_hbm.at[src_v], rows_v, sem).wait()
            pltpu.sync_copy(rows_v, agg_sh.at[dst_v], add=True)
            return carry

        lax.fori_loop(0, n_chunks, body, 0)
        plsc.subcore_barrier()
        pltpu.sync_copy(agg_sh.at[pl.ds(sid * rpt, rpt)],
                        out_hbm.at[pl.ds(cid * N + sid * rpt, rpt)])
        if tail:
            @pl.when(sid == 0)
            def _():
                pltpu.sync_copy(agg_sh.at[pl.ds(_NS * rpt, tail)],
                                out_hbm.at[pl.ds(cid * N + _NS * rpt, tail)])

    return sc_fn(m, src, dst, zrows)


def _proj_body(x_ref, wT_ref, b_ref, w0_ref, h_ref, m_ref):
    h = jnp.dot(x_ref[...], wT_ref[...], preferred_element_type=jnp.float32)
    h = jnp.maximum(h + b_ref[...], 0.0)
    h_ref[...] = h
    m_ref[...] = jnp.dot(h, w0_ref[...], preferred_element_type=jnp.float32)


def _gru_body(p0_ref, p1_ref, hh_ref, wihT_ref, whhT_ref, bih_ref, bhh_ref,
              wn_ref, lng_ref, lnb_ref, hin_ref, hh_out_ref, m_out_ref,
              *, H, do_ln, do_res, has_next):
    agg = p0_ref[...] + p1_ref[...]
    hh = hh_ref[...]
    gi = jnp.dot(agg, wihT_ref[...], preferred_element_type=jnp.float32)
    gi = gi + bih_ref[...]
    gh = jnp.dot(hh, whhT_ref[...], preferred_element_type=jnp.float32)
    gh = gh + bhh_ref[...]
    r = jax.nn.sigmoid(gi[:, :H] + gh[:, :H])
    z = jax.nn.sigmoid(gi[:, H:2 * H] + gh[:, H:2 * H])
    n = jnp.tanh(gi[:, 2 * H:] + r * gh[:, 2 * H:])
    out = (1.0 - z) * n + z * hh
    if do_ln:
        mu = jnp.mean(out, axis=-1, keepdims=True)
        var = jnp.mean((out - mu) ** 2, axis=-1, keepdims=True)
        out = (out - mu) / jnp.sqrt(var + 1e-5) * lng_ref[...] + lnb_ref[...]
        if do_res:
            out = out + hin_ref[...]
        out = jnp.maximum(out, 0.0)
    hh_out_ref[...] = out
    if has_next:
        m_out_ref[...] = jnp.dot(out, wn_ref[...],
                                 preferred_element_type=jnp.float32)


_ROWBLK = 2000


def _row_spec(R, H):
    return pl.BlockSpec((R, H), lambda i: (i, 0))


def _full_spec(shape):
    return pl.BlockSpec(shape, lambda i: tuple(0 for _ in shape))


def _tc_proj(x, in_wT, in_b2, w0):
    N, D = x.shape
    H = in_wT.shape[1]
    R = _ROWBLK
    return pl.pallas_call(
        _proj_body,
        grid=(N // R,),
        in_specs=[_row_spec(R, D), _full_spec((D, H)), _full_spec((1, H)),
                  _full_spec((H, H))],
        out_specs=(_row_spec(R, H), _row_spec(R, H)),
        out_shape=(jax.ShapeDtypeStruct((N, H), jnp.float32),
                   jax.ShapeDtypeStruct((N, H), jnp.float32)),
    )(x, in_wT, in_b2, w0)


def _tc_gru(p0, p1, hh, wihT, whhT, bih2, bhh2, wn, lng2, lnb2, hin,
            do_ln, do_res, has_next):
    N, H = hh.shape
    R = _ROWBLK
    body = functools.partial(_gru_body, H=H, do_ln=do_ln, do_res=do_res,
                             has_next=has_next)
    out_shape = (jax.ShapeDtypeStruct((N, H), jnp.float32),
                 jax.ShapeDtypeStruct((N, H), jnp.float32))
    in_specs = [_row_spec(R, H), _row_spec(R, H), _row_spec(R, H),
                _full_spec((H, 3 * H)), _full_spec((H, 3 * H)),
                _full_spec((1, 3 * H)), _full_spec((1, 3 * H)),
                _full_spec((H, H)), _full_spec((1, H)), _full_spec((1, H)),
                _row_spec(R, H)]
    return pl.pallas_call(
        body, grid=(N // R,), in_specs=in_specs,
        out_specs=(_row_spec(R, H), _row_spec(R, H)),
        out_shape=out_shape)(
        p0, p1, hh, wihT, whhT, bih2, bhh2, wn, lng2, lnb2, hin)


def kernel(x, edge_index, in_w, in_b, ggnn_w, wih, whh, bih, bhh, ln_g, ln_b):
    N, D = x.shape
    H = in_w.shape[0]
    L, S = ggnn_w.shape[0], ggnn_w.shape[1]
    perm = jnp.argsort(edge_index[1], stable=True)
    src = edge_index[0][perm]
    dst = edge_index[1][perm]
    zrows = jnp.zeros(((N // _NS) // 8 * 8, H), jnp.float32)
    wihT = jnp.transpose(wih, (0, 2, 1))
    whhT = jnp.transpose(whh, (0, 2, 1))
    bih2 = bih.reshape(L, 1, 3 * H)
    bhh2 = bhh.reshape(L, 1, 3 * H)
    lng2 = ln_g.reshape(L, 1, H)
    lnb2 = ln_b.reshape(L, 1, H)

    h, m = _tc_proj(x, in_w.T, in_b.reshape(1, H), ggnn_w[0, 0])
    for l in range(L):
        h_in = h
        hh = h
        for s in range(S):
            parts = _sc_aggregate(m, src, dst, zrows)
            p0 = parts[:N]
            p1 = parts[N:]
            last = s == S - 1
            if not last:
                hh, m = _tc_gru(p0, p1, hh, wihT[l], whhT[l], bih2[l],
                                bhh2[l], ggnn_w[l, s + 1], lng2[l], lnb2[l],
                                h_in, False, False, True)
            else:
                wn = ggnn_w[l + 1, 0] if l < L - 1 else ggnn_w[l, 0]
                h, m = _tc_gru(p0, p1, hh, wihT[l], whhT[l], bih2[l],
                               bhh2[l], wn, lng2[l], lnb2[l], h_in,
                               True, l > 0, l < L - 1)
    return h
